# baseline (speedup 1.0000x reference)
import functools

import jax
import jax.numpy as jnp
from jax import lax
from jax.experimental import pallas as pl
from jax.experimental.pallas import tpu as pltpu
from jax.experimental.pallas import tpu_sc as plsc

B, N, H, S = 8, 512, 16, 512
T = 128
TP = T + 1
ISUB = 8
NTI = N // T
TILES = B * NTI * NTI
NC, NS, L = 2, 16, 16
NW = NC * NS


HP = H // 2
REP = 4
_MASK_HI = -65536


def _decode(tile):
    b = tile // (NTI * NTI)
    rest = tile % (NTI * NTI)
    return b, (rest // NTI) * T, (rest % NTI) * T


def _tec_body(attn_hbm, pos_hbm, wf_hbm, wr_hbm, out_hbm,
              wf_v, wr_v, p1_v, p2_v, p2f_v, a_v, ob_v,
              sem0, sem1, semp2, semp1, sema):
    wid = lax.axis_index("s") * NC + lax.axis_index("c")
    pltpu.sync_copy(wf_hbm, wf_v)
    pltpu.sync_copy(wr_hbm, wr_v)
    iota = lax.iota(jnp.int32, L)
    iota_tp = iota * TP
    rlane = iota & (REP - 1)
    tiles_per = TILES // NW
    nsub = T // ISUB

    def f32(x):
        return plsc.bitcast(x, jnp.float32)

    def p2_copy(t):
        tile = wid * tiles_per + t
        b, i0, j0 = _decode(tile)
        return pltpu.make_async_copy(
            pos_hbm.at[b, pl.ds(j0, T), pl.ds(i0, T)], p2_v.at[t % 2], semp2)

    p2_copy(0).start()
    for t in range(tiles_per):
        tile = wid * tiles_per + t
        b, i0, j0 = _decode(tile)
        p2_copy(t).wait()
        if t + 1 < tiles_per:
            p2_copy(t + 1).start()
        p2t = p2_v.at[t % 2]

        @plsc.parallel_loop(0, T * (T // L), unroll=2)
        def repack(k):
            r = k >> 3
            c = (k & 7) * L
            p2f_v[pl.ds(r * TP + c, L)] = p2t[r, pl.ds(c, L)]

        def in_copies(sb, buf):
            row = i0 + sb * ISUB
            return (
                pltpu.make_async_copy(
                    pos_hbm.at[b, pl.ds(row, ISUB), pl.ds(j0, T)],
                    p1_v.at[buf], semp1),
                pltpu.make_async_copy(
                    attn_hbm.at[b, pl.ds(row, ISUB), pl.ds(j0, T)],
                    a_v.at[buf], sema),
            )

        for c in in_copies(0, 0):
            c.start()

        def out_slice(sb):
            return out_hbm.at[b, :, pl.ds(i0 + sb * ISUB, ISUB), pl.ds(j0, T)]

        def flush_wait(sb, buf, sem):
            pltpu.make_async_copy(ob_v.at[buf], out_slice(sb - 2), sem).wait()

        def isub_body(sb, carry):
            buf = lax.rem(sb, 2)
            nxt = lax.rem(sb + 1, 2)

            for c in in_copies(sb, buf):
                c.wait()

            @pl.when(sb + 1 < nsub)
            def _():
                for c in in_copies(sb + 1, nxt):
                    c.start()

            @pl.when(jnp.logical_and(sb >= 2, buf == 0))
            def _():
                flush_wait(sb, 0, sem0)

            @pl.when(jnp.logical_and(sb >= 2, buf == 1))
            def _():
                flush_wait(sb, 1, sem1)

            @plsc.parallel_loop(0, ISUB * (T // L), unroll=2)
            def ij_body(k):
                il = k >> 3
                jj = (k & 7) * L
                i = sb * ISUB + il
                p1seg = p1_v[buf, il, pl.ds(jj, L)]
                p2seg = plsc.load_gather(
                    p2f_v, [(jj * TP + i) + iota_tp])
                aseg = a_v[buf, il, pl.ds(jj, L)]
                p1r = (p1seg << 2) | rlane
                p2r = (p2seg << 2) | rlane
                g1s = [plsc.load_gather(
                    wf_v.at[pl.ds(hp * S * REP, S * REP)], [p1r])
                    for hp in range(HP)]
                g2s = [plsc.load_gather(
                    wr_v.at[pl.ds(hp * S * REP, S * REP)], [p2r])
                    for hp in range(HP)]
                for hp in range(HP):
                    g1, g2 = g1s[hp], g2s[hp]
                    lo = f32(g1 << 16) + f32(g2 << 16) + aseg
                    hi = f32(g1) + f32(g2) + aseg
                    ob_v[buf, 2 * hp, il, pl.ds(jj, L)] = lo
                    ob_v[buf, 2 * hp + 1, il, pl.ds(jj, L)] = hi

            @pl.when(buf == 0)
            def _():
                pltpu.async_copy(ob_v.at[0], out_slice(sb), sem0)

            @pl.when(buf == 1)
            def _():
                pltpu.async_copy(ob_v.at[1], out_slice(sb), sem1)

            return carry

        lax.fori_loop(0, nsub, isub_body, 0)
        flush_wait(nsub, 0, sem0)
        flush_wait(nsub + 1, 1, sem1)


@jax.jit
def _run(attn_bias, spatial_pos, wf, wr):
    mesh = plsc.VectorSubcoreMesh(core_axis_name="c", subcore_axis_name="s")
    kfn = functools.partial(
        pl.kernel,
        mesh=mesh,
        out_type=jax.ShapeDtypeStruct((B, H, N, N), jnp.float32),
        compiler_params=pltpu.CompilerParams(needs_layout_passes=False),
        scratch_types=[
            pltpu.VMEM((HP * S * REP,), jnp.int32),
            pltpu.VMEM((HP * S * REP,), jnp.int32),
            pltpu.VMEM((2, ISUB, T), jnp.int32),
            pltpu.VMEM((2, T, T), jnp.int32),
            pltpu.VMEM((T * TP,), jnp.int32),
            pltpu.VMEM((2, ISUB, T), jnp.float32),
            pltpu.VMEM((2, H, ISUB, T), jnp.float32),
            pltpu.SemaphoreType.DMA,
            pltpu.SemaphoreType.DMA,
            pltpu.SemaphoreType.DMA,
            pltpu.SemaphoreType.DMA,
            pltpu.SemaphoreType.DMA,
        ],
    )(_tec_body)
    return kfn(attn_bias, spatial_pos, wf, wr)


def _pack_table(w):
    bits = jax.lax.bitcast_convert_type(w.astype(jnp.float32), jnp.uint32)
    lsb = (bits >> 16) & 1
    hi16 = (bits + 0x7FFF + lsb) & jnp.uint32(0xFFFF0000)
    packed = (hi16[:, 0::2] >> 16) | hi16[:, 1::2]
    flat = jax.lax.bitcast_convert_type(
        jnp.transpose(packed), jnp.int32).reshape(-1)
    return jnp.repeat(flat, REP)


def kernel(attn_bias, spatial_pos, W_spatial, W_spatial_rev):
    return _run(attn_bias.astype(jnp.float32),
                spatial_pos.astype(jnp.int32),
                _pack_table(W_spatial), _pack_table(W_spatial_rev))


# speedup vs baseline: 1.0180x; 1.0180x over previous
import functools

import jax
import jax.numpy as jnp
from jax import lax
from jax.experimental import pallas as pl
from jax.experimental.pallas import tpu as pltpu
from jax.experimental.pallas import tpu_sc as plsc

B, N, H, S = 8, 512, 16, 512
T = 128
TP = T + 1
ISUB = 8
NTI = N // T
TILES = B * NTI * NTI
NC, NS, L = 2, 16, 16
NW = NC * NS


HP = H // 2
_MASK_HI = -65536


def _decode(tile):
    b = tile // (NTI * NTI)
    rest = tile % (NTI * NTI)
    return b, (rest // NTI) * T, (rest % NTI) * T


def _tec_body(attn_hbm, pos_hbm, wf_hbm, wr_hbm, out_hbm,
              wf_v, wr_v, p1_v, p2_v, p2f_v, a_v, ob_v,
              sem0, sem1, sem2, semp2, semp1, sema):
    wid = lax.axis_index("s") * NC + lax.axis_index("c")
    pltpu.sync_copy(wf_hbm, wf_v)
    pltpu.sync_copy(wr_hbm, wr_v)
    iota = lax.iota(jnp.int32, L)
    iota_tp = iota * TP
    tiles_per = TILES // NW
    nsub = T // ISUB

    def f32(x):
        return plsc.bitcast(x, jnp.float32)

    def p2_copy(t):
        tile = wid * tiles_per + t
        b, i0, j0 = _decode(tile)
        return pltpu.make_async_copy(
            pos_hbm.at[b, pl.ds(j0, T), pl.ds(i0, T)], p2_v.at[t % 2], semp2)

    p2_copy(0).start()
    for t in range(tiles_per):
        tile = wid * tiles_per + t
        b, i0, j0 = _decode(tile)
        p2_copy(t).wait()
        if t + 1 < tiles_per:
            p2_copy(t + 1).start()
        p2t = p2_v.at[t % 2]

        @plsc.parallel_loop(0, T * (T // L), unroll=2)
        def repack(k):
            r = k >> 3
            c = (k & 7) * L
            p2f_v[pl.ds(r * TP + c, L)] = p2t[r, pl.ds(c, L)]

        def in_copies(sb, buf):
            row = i0 + sb * ISUB
            return (
                pltpu.make_async_copy(
                    pos_hbm.at[b, pl.ds(row, ISUB), pl.ds(j0, T)],
                    p1_v.at[buf], semp1),
                pltpu.make_async_copy(
                    attn_hbm.at[b, pl.ds(row, ISUB), pl.ds(j0, T)],
                    a_v.at[buf], sema),
            )

        for c in in_copies(0, 0):
            c.start()

        def out_slice(sb):
            return out_hbm.at[b, :, pl.ds(i0 + sb * ISUB, ISUB), pl.ds(j0, T)]

        obsems = (sem0, sem1, sem2)

        def flush_wait(issue_sb, obuf):
            pltpu.make_async_copy(
                ob_v.at[obuf], out_slice(issue_sb), obsems[obuf]).wait()

        def isub_body(sb, carry):
            buf = lax.rem(sb, 2)
            nxt = lax.rem(sb + 1, 2)
            obuf = lax.rem(sb, 3)

            for c in in_copies(sb, buf):
                c.wait()

            @pl.when(sb + 1 < nsub)
            def _():
                for c in in_copies(sb + 1, nxt):
                    c.start()

            for k in range(3):
                @pl.when(jnp.logical_and(sb >= 3, obuf == k))
                def _(k=k):
                    flush_wait(sb - 3, k)

            @plsc.parallel_loop(0, ISUB * (T // L), unroll=2)
            def ij_body(k):
                il = k >> 3
                jj = (k & 7) * L
                i = sb * ISUB + il
                p1seg = p1_v[buf, il, pl.ds(jj, L)]
                p2seg = plsc.load_gather(
                    p2f_v, [(jj * TP + i) + iota_tp])
                aseg = a_v[buf, il, pl.ds(jj, L)]
                g1s = [plsc.load_gather(wf_v.at[pl.ds(hp * S, S)], [p1seg])
                       for hp in range(HP)]
                g2s = [plsc.load_gather(wr_v.at[pl.ds(hp * S, S)], [p2seg])
                       for hp in range(HP)]
                for hp in range(HP):
                    g1, g2 = g1s[hp], g2s[hp]
                    lo = f32(g1 << 16) + f32(g2 << 16) + aseg
                    hi = f32(g1) + f32(g2) + aseg
                    ob_v[obuf, 2 * hp, il, pl.ds(jj, L)] = lo
                    ob_v[obuf, 2 * hp + 1, il, pl.ds(jj, L)] = hi

            for k in range(3):
                @pl.when(obuf == k)
                def _(k=k):
                    pltpu.async_copy(ob_v.at[k], out_slice(sb), obsems[k])

            return carry

        lax.fori_loop(0, nsub, isub_body, 0)
        for sbq in (nsub - 3, nsub - 2, nsub - 1):
            flush_wait(sbq, sbq % 3)


@jax.jit
def _run(attn_bias, spatial_pos, wf, wr):
    mesh = plsc.VectorSubcoreMesh(core_axis_name="c", subcore_axis_name="s")
    kfn = functools.partial(
        pl.kernel,
        mesh=mesh,
        out_type=jax.ShapeDtypeStruct((B, H, N, N), jnp.float32),
        compiler_params=pltpu.CompilerParams(needs_layout_passes=False),
        scratch_types=[
            pltpu.VMEM((HP * S,), jnp.int32),
            pltpu.VMEM((HP * S,), jnp.int32),
            pltpu.VMEM((2, ISUB, T), jnp.int32),
            pltpu.VMEM((2, T, T), jnp.int32),
            pltpu.VMEM((T * TP,), jnp.int32),
            pltpu.VMEM((2, ISUB, T), jnp.float32),
            pltpu.VMEM((3, H, ISUB, T), jnp.float32),
            pltpu.SemaphoreType.DMA,
            pltpu.SemaphoreType.DMA,
            pltpu.SemaphoreType.DMA,
            pltpu.SemaphoreType.DMA,
            pltpu.SemaphoreType.DMA,
            pltpu.SemaphoreType.DMA,
        ],
    )(_tec_body)
    return kfn(attn_bias, spatial_pos, wf, wr)


def _pack_table(w):
    bits = jax.lax.bitcast_convert_type(w.astype(jnp.float32), jnp.uint32)
    lsb = (bits >> 16) & 1
    hi16 = (bits + 0x7FFF + lsb) & jnp.uint32(0xFFFF0000)
    packed = (hi16[:, 0::2] >> 16) | hi16[:, 1::2]
    return jax.lax.bitcast_convert_type(
        jnp.transpose(packed), jnp.int32).reshape(-1)


def kernel(attn_bias, spatial_pos, W_spatial, W_spatial_rev):
    return _run(attn_bias.astype(jnp.float32),
                spatial_pos.astype(jnp.int32),
                _pack_table(W_spatial), _pack_table(W_spatial_rev))


# speedup vs baseline: 1.0351x; 1.0167x over previous
import functools

import jax
import jax.numpy as jnp
from jax import lax
from jax.experimental import pallas as pl
from jax.experimental.pallas import tpu as pltpu
from jax.experimental.pallas import tpu_sc as plsc

B, N, H, S = 8, 512, 16, 512
T = 128
TP = T + 1
ISUB = 8
NTI = N // T
TILES = B * NTI * NTI
NC, NS, L = 2, 16, 16
NW = NC * NS


HP = H // 2
NOB = 2
_MASK_HI = -65536


def _decode(tile):
    b = tile // (NTI * NTI)
    rest = tile % (NTI * NTI)
    return b, (rest // NTI) * T, (rest % NTI) * T


def _tec_body(attn_hbm, pos_hbm, wf_hbm, wr_hbm, out_hbm,
              wf_v, wr_v, p1_v, p2_v, p2f_v, a_v, ob_v,
              sem0, sem1, sem2, semp2, semp1, sema):
    wid = lax.axis_index("s") * NC + lax.axis_index("c")
    pltpu.sync_copy(wf_hbm, wf_v)
    pltpu.sync_copy(wr_hbm, wr_v)
    iota = lax.iota(jnp.int32, L)
    iota_tp = iota * TP
    tiles_per = TILES // NW
    nsub = T // ISUB

    def f32(x):
        return plsc.bitcast(x, jnp.float32)

    def p2_copy(t):
        tile = wid * tiles_per + t
        b, i0, j0 = _decode(tile)
        return pltpu.make_async_copy(
            pos_hbm.at[b, pl.ds(j0, T), pl.ds(i0, T)], p2_v.at[t % 2], semp2)

    p2_copy(0).start()
    for t in range(tiles_per):
        tile = wid * tiles_per + t
        b, i0, j0 = _decode(tile)
        p2_copy(t).wait()
        if t + 1 < tiles_per:
            p2_copy(t + 1).start()
        p2t = p2_v.at[t % 2]

        @plsc.parallel_loop(0, T * (T // L), unroll=2)
        def repack(k):
            r = k >> 3
            c = (k & 7) * L
            p2f_v[pl.ds(r * TP + c, L)] = p2t[r, pl.ds(c, L)]

        def in_copies(sb, buf):
            row = i0 + sb * ISUB
            return (
                pltpu.make_async_copy(
                    pos_hbm.at[b, pl.ds(row, ISUB), pl.ds(j0, T)],
                    p1_v.at[buf], semp1),
                pltpu.make_async_copy(
                    attn_hbm.at[b, pl.ds(row, ISUB), pl.ds(j0, T)],
                    a_v.at[buf], sema),
            )

        for c in in_copies(0, 0):
            c.start()

        def out_slice(sb):
            return out_hbm.at[b, :, pl.ds(i0 + sb * ISUB, ISUB), pl.ds(j0, T)]

        obsems = (sem0, sem1, sem2)

        def flush_wait(issue_sb, obuf):
            pltpu.make_async_copy(
                ob_v.at[obuf], out_slice(issue_sb), obsems[obuf]).wait()

        def isub_body(sb, carry):
            buf = lax.rem(sb, 2)
            nxt = lax.rem(sb + 1, 2)
            obuf = lax.rem(sb, NOB)

            for c in in_copies(sb, buf):
                c.wait()

            @pl.when(sb + 1 < nsub)
            def _():
                for c in in_copies(sb + 1, nxt):
                    c.start()

            for k in range(NOB):
                @pl.when(jnp.logical_and(sb >= NOB, obuf == k))
                def _(k=k):
                    flush_wait(sb - NOB, k)

            @plsc.parallel_loop(0, ISUB * (T // L), unroll=2)
            def ij_body(k):
                il = k >> 3
                jj = (k & 7) * L
                i = sb * ISUB + il
                p1seg = p1_v[buf, il, pl.ds(jj, L)]
                p2seg = plsc.load_gather(
                    p2f_v, [(jj * TP + i) + iota_tp])
                aseg = a_v[buf, il, pl.ds(jj, L)]
                g1s = [plsc.load_gather(wf_v.at[pl.ds(hp * S, S)], [p1seg])
                       for hp in range(HP)]
                g2s = [plsc.load_gather(wr_v.at[pl.ds(hp * S, S)], [p2seg])
                       for hp in range(HP)]
                for hp in range(HP):
                    g1, g2 = g1s[hp], g2s[hp]
                    lo = f32(g1 << 16) + f32(g2 << 16) + aseg
                    hi = f32(g1) + f32(g2) + aseg
                    ob_v[obuf, 2 * hp, il, pl.ds(jj, L)] = lo
                    ob_v[obuf, 2 * hp + 1, il, pl.ds(jj, L)] = hi

            for k in range(NOB):
                @pl.when(obuf == k)
                def _(k=k):
                    pltpu.async_copy(ob_v.at[k], out_slice(sb), obsems[k])

            return carry

        lax.fori_loop(0, nsub, isub_body, 0)
        for sbq in range(nsub - NOB, nsub):
            flush_wait(sbq, sbq % NOB)


@jax.jit
def _run(attn_bias, spatial_pos, wf, wr):
    mesh = plsc.VectorSubcoreMesh(core_axis_name="c", subcore_axis_name="s")
    kfn = functools.partial(
        pl.kernel,
        mesh=mesh,
        out_type=jax.ShapeDtypeStruct((B, H, N, N), jnp.float32),
        compiler_params=pltpu.CompilerParams(needs_layout_passes=False),
        scratch_types=[
            pltpu.VMEM((HP * S,), jnp.int32),
            pltpu.VMEM((HP * S,), jnp.int32),
            pltpu.VMEM((2, ISUB, T), jnp.int32),
            pltpu.VMEM((2, T, T), jnp.int32),
            pltpu.VMEM((T * TP,), jnp.int32),
            pltpu.VMEM((2, ISUB, T), jnp.float32),
            pltpu.VMEM((NOB, H, ISUB, T), jnp.float32),
            pltpu.SemaphoreType.DMA,
            pltpu.SemaphoreType.DMA,
            pltpu.SemaphoreType.DMA,
            pltpu.SemaphoreType.DMA,
            pltpu.SemaphoreType.DMA,
            pltpu.SemaphoreType.DMA,
        ],
    )(_tec_body)
    return kfn(attn_bias, spatial_pos, wf, wr)


def _pack_table(w):
    bits = jax.lax.bitcast_convert_type(w.astype(jnp.float32), jnp.uint32)
    lsb = (bits >> 16) & 1
    hi16 = (bits + 0x7FFF + lsb) & jnp.uint32(0xFFFF0000)
    packed = (hi16[:, 0::2] >> 16) | hi16[:, 1::2]
    return jax.lax.bitcast_convert_type(
        jnp.transpose(packed), jnp.int32).reshape(-1)


def kernel(attn_bias, spatial_pos, W_spatial, W_spatial_rev):
    return _run(attn_bias.astype(jnp.float32),
                spatial_pos.astype(jnp.int32),
                _pack_table(W_spatial), _pack_table(W_spatial_rev))
